# baseline (speedup 1.0000x reference)
import functools

import jax
import jax.numpy as jnp
from jax import lax
from jax.experimental import pallas as pl
from jax.experimental.pallas import tpu as pltpu
from jax.experimental.pallas import tpu_sc as plsc

H = 128
T = 2
f32 = jnp.float32
i32 = jnp.int32


def _lrelu(v):
    return jnp.where(v > 0, v, 0.2 * v)


def _k1_body(src_ref, tgt_ref, st_ref, w_ref, a8_ref, b8_ref,
             sg1_ref, sgb1_ref, sg2_ref, sgb2_ref,
             tg1_ref, tgb1_ref, tg2_ref, tgb2_ref,
             tsi_ref, tti_ref, sgate_ref, tgate_ref, proj_ref):
    s = src_ref[...]
    t = tgt_ref[...]
    w = w_ref[...]
    tsi = jnp.dot(s, w, preferred_element_type=f32)
    tti = jnp.dot(t, w, preferred_element_type=f32)
    tsi_ref[...] = tsi
    tti_ref[...] = tti
    proj = (jnp.dot(tsi, a8_ref[...], preferred_element_type=f32)
            + jnp.dot(tti, b8_ref[...], preferred_element_type=f32))
    col = lax.broadcasted_iota(i32, (1, 8), 1)
    proj_ref[...] = jnp.where(col == 6, st_ref[...], proj)
    h1 = _lrelu(jnp.dot(t, sg1_ref[...], preferred_element_type=f32) + sgb1_ref[...])
    sgate_ref[...] = _lrelu(jnp.dot(h1, sg2_ref[...], preferred_element_type=f32) + sgb2_ref[...])
    h2 = _lrelu(jnp.dot(s, tg1_ref[...], preferred_element_type=f32) + tgb1_ref[...])
    tgate_ref[...] = _lrelu(jnp.dot(h2, tg2_ref[...], preferred_element_type=f32) + tgb2_ref[...])


def _k2_body(sums_ref, st_ref, seed_ref, par_ref, nst_ref, sinv_ref):
    smat = sums_ref[0] + sums_ref[1]
    sE = smat[0:1]
    sES = smat[1:2]
    sEW = smat[2:3]
    sFG = smat[3:5]
    sws = par_ref[:, 0:1]
    swn = par_ref[:, 1:2]
    swa = par_ref[:, 2:3]
    swe = par_ref[:, 3:4]
    a_v = swa * sES / (sE + 1e-16) + swe * sEW
    st = st_ref[...]
    seed = seed_ref[...]
    nst = jax.nn.sigmoid(st * sws + a_v * swn)
    nst_ref[...] = nst * (1.0 - seed) + seed
    sinv_ref[...] = 1.0 / (sFG + 1e-16)


def _k3_body(tsi_ref, tti_ref, bv_ref, cv_ref, nst_ref, par_ref, ns_ref, nt_ref):
    p = par_ref
    nst = nst_ref[...]
    ns_ref[...] = _lrelu(tsi_ref[...] * p[:, 0:1] + bv_ref[...] * p[:, 1:2] + nst * p[:, 2:3])
    nt_ref[...] = _lrelu(tti_ref[...] * p[:, 3:4] + cv_ref[...] * p[:, 4:5] + nst * p[:, 5:6])


def _kf_body(src_ref, tgt_ref, st_ref, th2_ref, th3_ref, th4_ref, th1_ref, o_ref):
    s = src_ref[...]
    t = tgt_ref[...]
    st = st_ref[...]
    f1 = _lrelu(jnp.dot(s, th2_ref[...], preferred_element_type=f32))
    f2 = _lrelu(jnp.dot(t, th3_ref[...], preferred_element_type=f32))
    f3 = _lrelu(jnp.dot(st * s, th4_ref[...], preferred_element_type=f32))
    th1 = th1_ref[...]
    o_ref[...] = (jnp.dot(f1, th1[:H], preferred_element_type=f32)
                  + jnp.dot(f2, th1[H:2 * H], preferred_element_type=f32)
                  + jnp.dot(f3, th1[2 * H:], preferred_element_type=f32))


def _make_passA(NPAD, G):
    RA = G // 32
    CHA = RA // 16
    mesh = plsc.VectorSubcoreMesh(core_axis_name="c", subcore_axis_name="s")

    @functools.partial(
        pl.kernel,
        out_type=(jax.ShapeDtypeStruct((2 * 5 * NPAD,), f32),
                  jax.ShapeDtypeStruct((2 * G, 128), f32)),
        mesh=mesh,
        compiler_params=pltpu.CompilerParams(needs_layout_passes=False),
        scratch_types=[
            pltpu.VMEM((NPAD * 8,), f32),
            pltpu.VMEM((16, 128), i32),
            pltpu.VMEM((16, 128), i32),
            pltpu.VMEM((16, 128), f32),
            pltpu.VMEM((16, 128), f32),
            pltpu.VMEM((16, 128), f32),
            pltpu.VMEM((16, 128), f32),
            pltpu.VMEM((16, 128), f32),
            pltpu.VMEM((16, 128), f32),
            pltpu.VMEM((NPAD,), f32),
            pltpu.VMEM_SHARED((NPAD,), f32),
            pltpu.VMEM_SHARED((NPAD,), f32),
            pltpu.VMEM_SHARED((NPAD,), f32),
            pltpu.VMEM_SHARED((NPAD,), f32),
            pltpu.VMEM_SHARED((NPAD,), f32),
            pltpu.SemaphoreType.DMA,
        ],
    )
    def passA(src_h, dst_h, ew_h, ns_h, sums_h, expfg_h,
              ns_v, src2, dst2, ew2, vale, vales, valew, valf, valg, stage,
              acc_e, acc_es, acc_ew, acc_f, acc_g, sem):
        c = lax.axis_index("c")
        s = lax.axis_index("s")
        accs = [acc_e, acc_es, acc_ew, acc_f, acc_g]

        def zb(k, carry):
            stage[pl.ds(k * 16, 16)] = jnp.zeros((16,), f32)
            return carry
        lax.fori_loop(0, NPAD // 16, zb, 0)
        for ai in range(5):
            @pl.when(s == ai)
            def _(ai=ai):
                pltpu.sync_copy(stage, accs[ai])
        plsc.subcore_barrier()

        pltpu.sync_copy(ns_h, ns_v)
        base = (c * 16 + s) * RA

        def chunk(ch, carry):
            rb = base + ch * 16
            pltpu.sync_copy(src_h.at[pl.ds(rb, 16), :], src2)
            pltpu.sync_copy(dst_h.at[pl.ds(rb, 16), :], dst2)
            pltpu.sync_copy(ew_h.at[pl.ds(rb, 16), :], ew2)

            def row(i, c2):
                for k in range(8):
                    sl = pl.ds(k * 16, 16)
                    si = src2[i, sl] * 8
                    di = dst2[i, sl] * 8
                    w = ew2[i, sl]
                    es = plsc.load_gather(ns_v, [si])
                    fs = plsc.load_gather(ns_v, [si + 1])
                    gs = plsc.load_gather(ns_v, [si + 2])
                    et = plsc.load_gather(ns_v, [di + 3])
                    ft = plsc.load_gather(ns_v, [di + 4])
                    gt = plsc.load_gather(ns_v, [di + 5])
                    st = plsc.load_gather(ns_v, [si + 6])
                    e = es + et
                    expe = jnp.exp(jnp.where(e > 0, e, 0.2 * e))
                    f = fs + ft
                    expf = jnp.exp(jnp.where(f > 0, f, 0.2 * f))
                    g = gs + gt
                    expg = jnp.exp(jnp.where(g > 0, g, 0.2 * g))
                    vale[i, sl] = expe
                    vales[i, sl] = expe * st
                    valew[i, sl] = w * st
                    valf[i, sl] = expf
                    valg[i, sl] = expg
                return c2
            lax.fori_loop(0, 16, row, 0)

            pltpu.sync_copy(valf, expfg_h.at[pl.ds(rb, 16), :])
            pltpu.sync_copy(valg, expfg_h.at[pl.ds(G + rb, 16), :])
            pend = []
            for i in range(16):
                pend.append([
                    pltpu.async_copy(vale.at[i], acc_e.at[dst2.at[i]], sem, add=True),
                    pltpu.async_copy(vales.at[i], acc_es.at[dst2.at[i]], sem, add=True),
                    pltpu.async_copy(valew.at[i], acc_ew.at[dst2.at[i]], sem, add=True),
                    pltpu.async_copy(valf.at[i], acc_f.at[src2.at[i]], sem, add=True),
                    pltpu.async_copy(valg.at[i], acc_g.at[dst2.at[i]], sem, add=True),
                ])
                if i >= 2:
                    for dd in pend[i - 2]:
                        dd.wait()
            for row_d in pend[14:]:
                for dd in row_d:
                    dd.wait()
            return carry
        lax.fori_loop(0, CHA, chunk, 0)
        plsc.subcore_barrier()
        for ai in range(5):
            @pl.when(s == ai)
            def _(ai=ai):
                pltpu.sync_copy(accs[ai], stage)
                pltpu.sync_copy(stage, sums_h.at[pl.ds((c * 5 + ai) * NPAD, NPAD)])

    return passA


def _make_passW(NPAD, G):
    RB = G // 16
    CHB = RB // 16
    mesh = plsc.VectorSubcoreMesh(core_axis_name="c", subcore_axis_name="s")

    @functools.partial(
        pl.kernel,
        out_type=jax.ShapeDtypeStruct((2 * G * 128,), f32),
        mesh=mesh,
        compiler_params=pltpu.CompilerParams(needs_layout_passes=False),
        scratch_types=[
            pltpu.VMEM((NPAD,), f32),
            pltpu.VMEM((32, 64), i32),
            pltpu.VMEM((16, 128), f32),
            pltpu.VMEM((16, 128), f32),
            pltpu.VMEM((2048,), f32),
            pltpu.VMEM((16,), f32),
            pltpu.VMEM((16,), f32),
        ],
    )
    def passW(sidx_h, ew_h, exp_h, sinv_h, wa_h, wb_h, alph_h,
              sinv_v, sidx2, ew2, exp2, alpha_f, wa_v, wb_v):
        c = lax.axis_index("c")
        s = lax.axis_index("s")
        pltpu.sync_copy(wa_h.at[pl.ds(c * 16, 16)], wa_v)
        pltpu.sync_copy(wb_h.at[pl.ds(c * 16, 16)], wb_v)
        pltpu.sync_copy(sinv_h.at[pl.ds(c * NPAD, NPAD)], sinv_v)
        wa = wa_v[...]
        wb = wb_v[...]
        base = s * RB
        cbase = c * G + base

        def chunk0(ch, carry):
            pltpu.sync_copy(ew_h.at[pl.ds(base + ch * 16, 16), :], ew2)
            pltpu.sync_copy(exp_h.at[pl.ds(cbase + ch * 16, 16), :], exp2)
            pltpu.sync_copy(sidx_h.at[pl.ds((cbase + ch * 16) * 2, 32), :],
                            sidx2)

            def arow(i, c2):
                for k in range(8):
                    sl = pl.ds(k * 16, 16)
                    sg = plsc.load_gather(
                        sinv_v, [sidx2[2 * i + k // 4, pl.ds(16 * (k % 4), 16)]])
                    alpha_f[pl.ds(i * 128 + k * 16, 16)] = (
                        exp2[i, sl] * sg * wa + ew2[i, sl] * wb)
                return c2
            lax.fori_loop(0, 16, arow, 0)
            pltpu.sync_copy(alpha_f,
                            alph_h.at[pl.ds((cbase + ch * 16) * 128, 2048)])
            return carry
        lax.fori_loop(0, CHB, chunk0, 0)

    return passW

def _make_passB(NPAD, G):
    RB = G // 16
    CHB = RB // 16
    ZR = NPAD // 16
    ZFULL, ZREM = ZR // 128, ZR % 128
    mesh = plsc.VectorSubcoreMesh(core_axis_name="c", subcore_axis_name="s")

    @functools.partial(
        pl.kernel,
        out_type=jax.ShapeDtypeStruct((2, NPAD, H), f32),
        mesh=mesh,
        compiler_params=pltpu.CompilerParams(needs_layout_passes=False),
        scratch_types=[
            pltpu.VMEM((32, 64), i32),
            pltpu.VMEM((32, 64), i32),
            pltpu.VMEM((2048,), f32),
            pltpu.VMEM((64, H), f32),
            pltpu.VMEM((64, H), f32),
            pltpu.VMEM((64, H), f32),
            pltpu.VMEM((64, H), f32),
            pltpu.VMEM_SHARED((NPAD, H), f32),
            pltpu.SemaphoreType.DMA,
            pltpu.SemaphoreType.DMA,
            pltpu.SemaphoreType.DMA,
            pltpu.SemaphoreType.DMA,
            pltpu.SemaphoreType.DMA,
            pltpu.SemaphoreType.DMA,
            pltpu.SemaphoreType.DMA,
            pltpu.SemaphoreType.DMA,
        ],
    )
    def passB(gidx_h, sidx_h, alph_h, gate_h, out_h,
              gidx2, sidx2, alpha_f, rows0, rows1, rows2, rows3, acc,
              sg0, sg1, sg2, sg3, ss0, ss1, ss2, ss3):
        c = lax.axis_index("c")
        s = lax.axis_index("s")
        rows = [rows0, rows1, rows2, rows3]
        sem_g = [sg0, sg1, sg2, sg3]
        sem_s = [ss0, ss1, ss2, ss3]
        Z64 = ZR // 64
        Z64R = ZR % 64

        def zr(k, carry):
            for v in range(8):
                rows0[k, pl.ds(v * 16, 16)] = jnp.zeros((16,), f32)
            return carry
        lax.fori_loop(0, 64, zr, 0)
        zbase = s * ZR
        for kk in range(Z64):
            pltpu.sync_copy(rows0, acc.at[pl.ds(zbase + kk * 64, 64), :])
        if Z64R:
            pltpu.sync_copy(rows0.at[pl.ds(0, Z64R), :],
                            acc.at[pl.ds(zbase + Z64 * 64, Z64R), :])
        plsc.subcore_barrier()

        base = s * RB
        cbase = c * G + base

        bidx = [jnp.full((16,), q, i32) for q in range(16)]

        def scale(buf, t):
            def grp(g, carry):
                v = alpha_f[pl.ds(t * 64 + g * 16, 16)]
                for q in range(16):
                    av = v.at[bidx[q]].get(mode="promise_in_bounds")
                    r = g * 16 + q
                    for vv in range(8):
                        slv = pl.ds(vv * 16, 16)
                        buf[r, slv] = buf[r, slv] * av
                return carry
            lax.fori_loop(0, 4, grp, 0)

        def gfire(t, b):
            pltpu.async_copy(gate_h.at[gidx2.at[t]], rows[b], sem_g[b])

        def gwait(t, b):
            pltpu.make_async_copy(
                gate_h.at[gidx2.at[t]], rows[b], sem_g[b]).wait()

        def sfire(t, b):
            pltpu.async_copy(rows[b], acc.at[sidx2.at[t]], sem_s[b],
                             add=True)

        def swait(b):
            pltpu.make_async_copy(
                rows[b], acc.at[sidx2.at[0]], sem_s[b]).wait()

        def chunk1(ch, carry):
            hb2 = (cbase + ch * 16) * 2
            pltpu.sync_copy(gidx_h.at[pl.ds(hb2, 32), :], gidx2)
            pltpu.sync_copy(sidx_h.at[pl.ds(hb2, 32), :], sidx2)
            pltpu.sync_copy(alph_h.at[pl.ds((cbase + ch * 16) * 128, 2048)],
                            alpha_f)
            gfire(0, 0)
            gfire(1, 1)

            def quad(tq, c2):
                t0 = tq * 4
                for b in range(4):
                    t = t0 + b

                    @pl.when((t >= 2) | (ch > 0))
                    def _():
                        swait((b + 2) % 4)

                    @pl.when(t + 2 < 32)
                    def _():
                        gfire(t + 2, (b + 2) % 4)
                    gwait(t, b)
                    scale(rows[b], t)
                    sfire(t, b)
                return c2
            lax.fori_loop(0, 8, quad, 0)
            return carry
        lax.fori_loop(0, CHB, chunk1, 0)
        swait(2)
        swait(3)

        plsc.subcore_barrier()
        for kk in range(Z64):
            pltpu.sync_copy(acc.at[pl.ds(zbase + kk * 64, 64), :], rows0)
            pltpu.sync_copy(rows0, out_h.at[c, pl.ds(zbase + kk * 64, 64), :])
        if Z64R:
            pltpu.sync_copy(acc.at[pl.ds(zbase + Z64 * 64, Z64R), :],
                            rows0.at[pl.ds(0, Z64R), :])
            pltpu.sync_copy(rows0.at[pl.ds(0, Z64R), :],
                            out_h.at[c, pl.ds(zbase + Z64 * 64, Z64R), :])

    return passB


def kernel(x, edge_index, edge_weight, trans_W, eta, sws, swn, swa, swe, beta, sgW1, sgb1, sgW2, sgb2, s_ws, s_wn, s_wst, s_wa, s_we, tau, tgW1, tgb1, tgW2, tgb2, t_ws, t_wn, t_wst, t_wa, t_we, theta1, theta2, theta3, theta4):
    n = x.shape[0]
    e = edge_index.shape[1]
    NPAD = ((n + 1 + 127) // 128) * 128
    EP = ((e + 65535) // 65536) * 65536
    G = EP // 128
    NB = 10
    BR = n // NB

    src = edge_index[0]
    dst = edge_index[1]
    seed = x[:, -1]
    source = x[:, :H]
    target = x[:, H:2 * H]

    pad_i = jnp.full((EP - e,), n, i32)
    srcR = jnp.concatenate([src, pad_i]).reshape(G, 128)
    dstR = jnp.concatenate([dst, pad_i]).reshape(G, 128)
    ewR = jnp.concatenate([edge_weight, jnp.zeros((EP - e,), f32)]).reshape(G, 128)

    passA = _make_passA(NPAD, G)
    passW = _make_passW(NPAD, G)
    passB = _make_passB(NPAD, G)

    k1 = pl.pallas_call(
        _k1_body,
        out_shape=(jax.ShapeDtypeStruct((n, H), f32),) * 4
        + (jax.ShapeDtypeStruct((n, 8), f32),),
        grid=(NB,),
        in_specs=[
            pl.BlockSpec((BR, H), lambda i: (i, 0)),
            pl.BlockSpec((BR, H), lambda i: (i, 0)),
            pl.BlockSpec((BR, 1), lambda i: (i, 0)),
        ] + [pl.BlockSpec((H, H), lambda i: (0, 0)),
             pl.BlockSpec((H, 8), lambda i: (0, 0)),
             pl.BlockSpec((H, 8), lambda i: (0, 0)),
             pl.BlockSpec((H, H), lambda i: (0, 0)),
             pl.BlockSpec((1, H), lambda i: (0, 0)),
             pl.BlockSpec((H, H), lambda i: (0, 0)),
             pl.BlockSpec((1, H), lambda i: (0, 0)),
             pl.BlockSpec((H, H), lambda i: (0, 0)),
             pl.BlockSpec((1, H), lambda i: (0, 0)),
             pl.BlockSpec((H, H), lambda i: (0, 0)),
             pl.BlockSpec((1, H), lambda i: (0, 0))],
        out_specs=[pl.BlockSpec((BR, H), lambda i: (i, 0))] * 4
        + [pl.BlockSpec((BR, 8), lambda i: (i, 0))],
    )

    k2 = pl.pallas_call(
        _k2_body,
        out_shape=(jax.ShapeDtypeStruct((1, NPAD), f32),
                   jax.ShapeDtypeStruct((2, NPAD), f32)),
    )

    k3 = pl.pallas_call(
        _k3_body,
        out_shape=(jax.ShapeDtypeStruct((n, H), f32),) * 2,
        grid=(NB,),
        in_specs=[pl.BlockSpec((BR, H), lambda i: (i, 0))] * 4
        + [pl.BlockSpec((BR, 1), lambda i: (i, 0)),
           pl.BlockSpec((1, 8), lambda i: (0, 0))],
        out_specs=[pl.BlockSpec((BR, H), lambda i: (i, 0))] * 2,
    )

    kf = pl.pallas_call(
        _kf_body,
        out_shape=jax.ShapeDtypeStruct((n, 1), f32),
        grid=(NB,),
        in_specs=[
            pl.BlockSpec((BR, H), lambda i: (i, 0)),
            pl.BlockSpec((BR, H), lambda i: (i, 0)),
            pl.BlockSpec((BR, 1), lambda i: (i, 0)),
        ] + [pl.BlockSpec((H, H), lambda i: (0, 0)),
             pl.BlockSpec((H, H), lambda i: (0, 0)),
             pl.BlockSpec((H, H), lambda i: (0, 0)),
             pl.BlockSpec((3 * H, 1), lambda i: (0, 0))],
        out_specs=pl.BlockSpec((BR, 1), lambda i: (i, 0)),
    )

    state = seed
    zrow = jnp.zeros((NPAD - n, H), f32)
    seed_p = jnp.concatenate([seed, jnp.zeros((NPAD - n,), f32)]).reshape(1, NPAD)

    for i in range(T):
        a8 = jnp.concatenate(
            [eta[i][:H], beta[i][:H], tau[i][:H], jnp.zeros((H, 5), f32)], axis=1)
        b8 = jnp.concatenate(
            [jnp.zeros((H, 3), f32), eta[i][H:], beta[i][H:], tau[i][H:],
             jnp.zeros((H, 2), f32)], axis=1)
        tsi, tti, sgate, tgate, proj = k1(
            source, target, state[:, None], trans_W[i], a8, b8,
            sgW1[i].T, sgb1[i][None, :], sgW2[i].T, sgb2[i][None, :],
            tgW1[i].T, tgb1[i][None, :], tgW2[i].T, tgb2[i][None, :])

        ns = jnp.concatenate([proj, jnp.zeros((NPAD - n, 8), f32)], axis=0).reshape(NPAD * 8)
        sums, expfgR = passA(srcR, dstR, ewR, ns)
        sums = sums.reshape(2, 5, NPAD)

        par2 = jnp.stack([sws[i], swn[i], swa[i], swe[i],
                          jnp.zeros((), f32), jnp.zeros((), f32),
                          jnp.zeros((), f32), jnp.zeros((), f32)])[None, :]
        state_p = jnp.concatenate([state, jnp.zeros((NPAD - n,), f32)]).reshape(1, NPAD)
        nst, sinv = k2(sums, state_p, seed_p, par2)

        wa2 = jnp.concatenate([jnp.full((16,), s_wa[i], f32), jnp.full((16,), t_wa[i], f32)])
        wb2 = jnp.concatenate([jnp.full((16,), s_we[i], f32), jnp.full((16,), t_we[i], f32)])
        gidxM = jnp.concatenate([dstR, srcR + NPAD], axis=0).reshape(-1, 64)
        sidxM = jnp.concatenate([srcR, dstR], axis=0).reshape(-1, 64)
        gateM = jnp.concatenate([sgate, zrow, tgate, zrow], axis=0)
        alph = passW(sidxM, ewR, expfgR, sinv.reshape(2 * NPAD), wa2, wb2)
        bc = passB(gidxM, sidxM, alph, gateM)

        new_state = nst[0, :n]
        par3 = jnp.stack([s_ws[i], s_wn[i], s_wst[i],
                          t_ws[i], t_wn[i], t_wst[i],
                          jnp.zeros((), f32), jnp.zeros((), f32)])[None, :]
        source, target = k3(tsi, tti, bc[0, :n], bc[1, :n],
                            new_state[:, None], par3)
        state = new_state

    q = kf(source, target, state[:, None], theta2, theta3, theta4, theta1)
    return q[:, 0]


# speedup vs baseline: 1.0046x; 1.0046x over previous
import functools

import jax
import jax.numpy as jnp
from jax import lax
from jax.experimental import pallas as pl
from jax.experimental.pallas import tpu as pltpu
from jax.experimental.pallas import tpu_sc as plsc

H = 128
T = 2
f32 = jnp.float32
i32 = jnp.int32


def _lrelu(v):
    return jnp.where(v > 0, v, 0.2 * v)


def _k1_body(src_ref, tgt_ref, st_ref, w_ref, a8_ref, b8_ref,
             sg1_ref, sgb1_ref, sg2_ref, sgb2_ref,
             tg1_ref, tgb1_ref, tg2_ref, tgb2_ref,
             tsi_ref, tti_ref, sgate_ref, tgate_ref, proj_ref):
    s = src_ref[...]
    t = tgt_ref[...]
    w = w_ref[...]
    tsi = jnp.dot(s, w, preferred_element_type=f32)
    tti = jnp.dot(t, w, preferred_element_type=f32)
    tsi_ref[...] = tsi
    tti_ref[...] = tti
    proj = (jnp.dot(tsi, a8_ref[...], preferred_element_type=f32)
            + jnp.dot(tti, b8_ref[...], preferred_element_type=f32))
    col = lax.broadcasted_iota(i32, (1, 8), 1)
    proj_ref[...] = jnp.where(col == 6, st_ref[...], proj)
    h1 = _lrelu(jnp.dot(t, sg1_ref[...], preferred_element_type=f32) + sgb1_ref[...])
    sgate_ref[...] = _lrelu(jnp.dot(h1, sg2_ref[...], preferred_element_type=f32) + sgb2_ref[...])
    h2 = _lrelu(jnp.dot(s, tg1_ref[...], preferred_element_type=f32) + tgb1_ref[...])
    tgate_ref[...] = _lrelu(jnp.dot(h2, tg2_ref[...], preferred_element_type=f32) + tgb2_ref[...])


def _k2_body(sums_ref, st_ref, seed_ref, par_ref, nst_ref, sinv_ref):
    smat = sums_ref[0] + sums_ref[1]
    sE = smat[0:1]
    sES = smat[1:2]
    sEW = smat[2:3]
    sFG = smat[3:5]
    sws = par_ref[:, 0:1]
    swn = par_ref[:, 1:2]
    swa = par_ref[:, 2:3]
    swe = par_ref[:, 3:4]
    a_v = swa * sES / (sE + 1e-16) + swe * sEW
    st = st_ref[...]
    seed = seed_ref[...]
    nst = jax.nn.sigmoid(st * sws + a_v * swn)
    nst_ref[...] = nst * (1.0 - seed) + seed
    sinv_ref[...] = 1.0 / (sFG + 1e-16)


def _k3_body(tsi_ref, tti_ref, bv_ref, cv_ref, nst_ref, par_ref, ns_ref, nt_ref):
    p = par_ref
    nst = nst_ref[...]
    bv = bv_ref[...].astype(f32)
    cv = cv_ref[...].astype(f32)
    ns_ref[...] = _lrelu(tsi_ref[...] * p[:, 0:1] + bv * p[:, 1:2] + nst * p[:, 2:3])
    nt_ref[...] = _lrelu(tti_ref[...] * p[:, 3:4] + cv * p[:, 4:5] + nst * p[:, 5:6])


def _kf_body(src_ref, tgt_ref, st_ref, th2_ref, th3_ref, th4_ref, th1_ref, o_ref):
    s = src_ref[...]
    t = tgt_ref[...]
    st = st_ref[...]
    f1 = _lrelu(jnp.dot(s, th2_ref[...], preferred_element_type=f32))
    f2 = _lrelu(jnp.dot(t, th3_ref[...], preferred_element_type=f32))
    f3 = _lrelu(jnp.dot(st * s, th4_ref[...], preferred_element_type=f32))
    th1 = th1_ref[...]
    o_ref[...] = (jnp.dot(f1, th1[:H], preferred_element_type=f32)
                  + jnp.dot(f2, th1[H:2 * H], preferred_element_type=f32)
                  + jnp.dot(f3, th1[2 * H:], preferred_element_type=f32))


def _make_passA(NPAD, G):
    RA = G // 32
    CHA = RA // 16
    mesh = plsc.VectorSubcoreMesh(core_axis_name="c", subcore_axis_name="s")

    @functools.partial(
        pl.kernel,
        out_type=(jax.ShapeDtypeStruct((2 * 5 * NPAD,), f32),
                  jax.ShapeDtypeStruct((2 * G, 128), f32)),
        mesh=mesh,
        compiler_params=pltpu.CompilerParams(needs_layout_passes=False),
        scratch_types=[
            pltpu.VMEM((NPAD * 8,), f32),
            pltpu.VMEM((16, 128), i32),
            pltpu.VMEM((16, 128), i32),
            pltpu.VMEM((16, 128), f32),
            pltpu.VMEM((16, 128), f32),
            pltpu.VMEM((16, 128), f32),
            pltpu.VMEM((16, 128), f32),
            pltpu.VMEM((16, 128), f32),
            pltpu.VMEM((16, 128), f32),
            pltpu.VMEM((NPAD,), f32),
            pltpu.VMEM_SHARED((NPAD,), f32),
            pltpu.VMEM_SHARED((NPAD,), f32),
            pltpu.VMEM_SHARED((NPAD,), f32),
            pltpu.VMEM_SHARED((NPAD,), f32),
            pltpu.VMEM_SHARED((NPAD,), f32),
            pltpu.SemaphoreType.DMA,
        ],
    )
    def passA(src_h, dst_h, ew_h, ns_h, sums_h, expfg_h,
              ns_v, src2, dst2, ew2, vale, vales, valew, valf, valg, stage,
              acc_e, acc_es, acc_ew, acc_f, acc_g, sem):
        c = lax.axis_index("c")
        s = lax.axis_index("s")
        accs = [acc_e, acc_es, acc_ew, acc_f, acc_g]

        def zb(k, carry):
            stage[pl.ds(k * 16, 16)] = jnp.zeros((16,), f32)
            return carry
        lax.fori_loop(0, NPAD // 16, zb, 0)
        for ai in range(5):
            @pl.when(s == ai)
            def _(ai=ai):
                pltpu.sync_copy(stage, accs[ai])
        plsc.subcore_barrier()

        pltpu.sync_copy(ns_h, ns_v)
        base = (c * 16 + s) * RA

        def chunk(ch, carry):
            rb = base + ch * 16
            pltpu.sync_copy(src_h.at[pl.ds(rb, 16), :], src2)
            pltpu.sync_copy(dst_h.at[pl.ds(rb, 16), :], dst2)
            pltpu.sync_copy(ew_h.at[pl.ds(rb, 16), :], ew2)

            def row(i, c2):
                for k in range(8):
                    sl = pl.ds(k * 16, 16)
                    si = src2[i, sl] * 8
                    di = dst2[i, sl] * 8
                    w = ew2[i, sl]
                    es = plsc.load_gather(ns_v, [si])
                    fs = plsc.load_gather(ns_v, [si + 1])
                    gs = plsc.load_gather(ns_v, [si + 2])
                    et = plsc.load_gather(ns_v, [di + 3])
                    ft = plsc.load_gather(ns_v, [di + 4])
                    gt = plsc.load_gather(ns_v, [di + 5])
                    st = plsc.load_gather(ns_v, [si + 6])
                    e = es + et
                    expe = jnp.exp(jnp.where(e > 0, e, 0.2 * e))
                    f = fs + ft
                    expf = jnp.exp(jnp.where(f > 0, f, 0.2 * f))
                    g = gs + gt
                    expg = jnp.exp(jnp.where(g > 0, g, 0.2 * g))
                    vale[i, sl] = expe
                    vales[i, sl] = expe * st
                    valew[i, sl] = w * st
                    valf[i, sl] = expf
                    valg[i, sl] = expg
                return c2
            lax.fori_loop(0, 16, row, 0)

            pltpu.sync_copy(valf, expfg_h.at[pl.ds(rb, 16), :])
            pltpu.sync_copy(valg, expfg_h.at[pl.ds(G + rb, 16), :])
            pend = []
            for i in range(16):
                pend.append([
                    pltpu.async_copy(vale.at[i], acc_e.at[dst2.at[i]], sem, add=True),
                    pltpu.async_copy(vales.at[i], acc_es.at[dst2.at[i]], sem, add=True),
                    pltpu.async_copy(valew.at[i], acc_ew.at[dst2.at[i]], sem, add=True),
                    pltpu.async_copy(valf.at[i], acc_f.at[src2.at[i]], sem, add=True),
                    pltpu.async_copy(valg.at[i], acc_g.at[dst2.at[i]], sem, add=True),
                ])
                if i >= 2:
                    for dd in pend[i - 2]:
                        dd.wait()
            for row_d in pend[14:]:
                for dd in row_d:
                    dd.wait()
            return carry
        lax.fori_loop(0, CHA, chunk, 0)
        plsc.subcore_barrier()
        for ai in range(5):
            @pl.when(s == ai)
            def _(ai=ai):
                pltpu.sync_copy(accs[ai], stage)
                pltpu.sync_copy(stage, sums_h.at[pl.ds((c * 5 + ai) * NPAD, NPAD)])

    return passA


def _make_passW(NPAD, G):
    RB = G // 16
    CHB = RB // 16
    mesh = plsc.VectorSubcoreMesh(core_axis_name="c", subcore_axis_name="s")

    @functools.partial(
        pl.kernel,
        out_type=jax.ShapeDtypeStruct((2 * G * 128,), f32),
        mesh=mesh,
        compiler_params=pltpu.CompilerParams(needs_layout_passes=False),
        scratch_types=[
            pltpu.VMEM((NPAD,), f32),
            pltpu.VMEM((32, 64), i32),
            pltpu.VMEM((16, 128), f32),
            pltpu.VMEM((16, 128), f32),
            pltpu.VMEM((2048,), f32),
            pltpu.VMEM((16,), f32),
            pltpu.VMEM((16,), f32),
        ],
    )
    def passW(sidx_h, ew_h, exp_h, sinv_h, wa_h, wb_h, alph_h,
              sinv_v, sidx2, ew2, exp2, alpha_f, wa_v, wb_v):
        c = lax.axis_index("c")
        s = lax.axis_index("s")
        pltpu.sync_copy(wa_h.at[pl.ds(c * 16, 16)], wa_v)
        pltpu.sync_copy(wb_h.at[pl.ds(c * 16, 16)], wb_v)
        pltpu.sync_copy(sinv_h.at[pl.ds(c * NPAD, NPAD)], sinv_v)
        wa = wa_v[...]
        wb = wb_v[...]
        base = s * RB
        cbase = c * G + base

        def chunk0(ch, carry):
            pltpu.sync_copy(ew_h.at[pl.ds(base + ch * 16, 16), :], ew2)
            pltpu.sync_copy(exp_h.at[pl.ds(cbase + ch * 16, 16), :], exp2)
            pltpu.sync_copy(sidx_h.at[pl.ds((cbase + ch * 16) * 2, 32), :],
                            sidx2)

            def arow(i, c2):
                for k in range(8):
                    sl = pl.ds(k * 16, 16)
                    sg = plsc.load_gather(
                        sinv_v, [sidx2[2 * i + k // 4, pl.ds(16 * (k % 4), 16)]])
                    alpha_f[pl.ds(i * 128 + k * 16, 16)] = (
                        exp2[i, sl] * sg * wa + ew2[i, sl] * wb)
                return c2
            lax.fori_loop(0, 16, arow, 0)
            pltpu.sync_copy(alpha_f,
                            alph_h.at[pl.ds((cbase + ch * 16) * 128, 2048)])
            return carry
        lax.fori_loop(0, CHB, chunk0, 0)

    return passW

def _make_passB(NPAD, G):
    RB = G // 16
    CHB = RB // 16
    ZR = NPAD // 16
    ZFULL, ZREM = ZR // 128, ZR % 128
    mesh = plsc.VectorSubcoreMesh(core_axis_name="c", subcore_axis_name="s")

    @functools.partial(
        pl.kernel,
        out_type=jax.ShapeDtypeStruct((2, NPAD, H), f32),
        mesh=mesh,
        compiler_params=pltpu.CompilerParams(needs_layout_passes=False),
        scratch_types=[
            pltpu.VMEM((32, 64), i32),
            pltpu.VMEM((32, 64), i32),
            pltpu.VMEM((2048,), f32),
            pltpu.VMEM((64, H), f32),
            pltpu.VMEM((64, H), f32),
            pltpu.VMEM((64, H), f32),
            pltpu.VMEM((64, H), f32),
            pltpu.VMEM_SHARED((NPAD, H), f32),
            pltpu.SemaphoreType.DMA,
            pltpu.SemaphoreType.DMA,
            pltpu.SemaphoreType.DMA,
            pltpu.SemaphoreType.DMA,
            pltpu.SemaphoreType.DMA,
            pltpu.SemaphoreType.DMA,
            pltpu.SemaphoreType.DMA,
            pltpu.SemaphoreType.DMA,
        ],
    )
    def passB(gidx_h, sidx_h, alph_h, gate_h, out_h,
              gidx2, sidx2, alpha_f, g0, g1, g2, g3, acc,
              sg0, sg1, sg2, sg3, ss0, ss1, ss2, ss3):
        c = lax.axis_index("c")
        s = lax.axis_index("s")
        gbuf = [g0, g1, g2, g3]
        bbuf = gbuf
        sem_g = [sg0, sg1, sg2, sg3]
        sem_s = [ss0, ss1, ss2, ss3]
        ZR = NPAD // 16
        Z64 = ZR // 64

        def zr(k, carry):
            for v in range(8):
                g0[k, pl.ds(v * 16, 16)] = jnp.zeros((16,), f32)
            return carry
        lax.fori_loop(0, 64, zr, 0)
        zbase = s * ZR
        for kk in range(Z64):
            pltpu.sync_copy(g0, acc.at[pl.ds(zbase + kk * 64, 64), :])
        plsc.subcore_barrier()

        base = s * RB
        cbase = c * G + base

        bidx = [jnp.full((16,), q, i32) for q in range(16)]

        def scale(gb, bb, t):
            def grp(g, carry):
                v = alpha_f[pl.ds(t * 64 + g * 16, 16)]
                for q in range(16):
                    av = v.at[bidx[q]].get(mode="promise_in_bounds")
                    r = g * 16 + q
                    for vv in range(8):
                        slv = pl.ds(vv * 16, 16)
                        gb[r, slv] = gb[r, slv] * av
                return carry
            lax.fori_loop(0, 4, grp, 0)

        def gfire(t, b):
            pltpu.async_copy(gate_h.at[gidx2.at[t]], gbuf[b], sem_g[b])

        def gwait(t, b):
            pltpu.make_async_copy(
                gate_h.at[gidx2.at[t]], gbuf[b], sem_g[b]).wait()

        def sfire(t, b):
            pltpu.async_copy(bbuf[b], acc.at[sidx2.at[t]], sem_s[b],
                             add=True)

        def swait(b):
            pltpu.make_async_copy(
                bbuf[b], acc.at[sidx2.at[0]], sem_s[b]).wait()

        def chunk1(ch, carry):
            hb2 = (cbase + ch * 16) * 2
            pltpu.sync_copy(gidx_h.at[pl.ds(hb2, 32), :], gidx2)
            pltpu.sync_copy(sidx_h.at[pl.ds(hb2, 32), :], sidx2)
            pltpu.sync_copy(alph_h.at[pl.ds((cbase + ch * 16) * 128, 2048)],
                            alpha_f)
            gfire(0, 0)
            gfire(1, 1)

            def quad(tq, c2):
                t0 = tq * 4
                for b in range(4):
                    t = t0 + b

                    @pl.when((t >= 2) | (ch > 0))
                    def _():
                        swait((b + 2) % 4)

                    @pl.when(t + 2 < 32)
                    def _():
                        gfire(t + 2, (b + 2) % 4)
                    gwait(t, b)
                    scale(gbuf[b], bbuf[b], t)
                    sfire(t, b)
                return c2
            lax.fori_loop(0, 8, quad, 0)
            return carry
        lax.fori_loop(0, CHB, chunk1, 0)
        swait(2)
        swait(3)

        plsc.subcore_barrier()
        for kk in range(Z64):
            pltpu.sync_copy(acc.at[pl.ds(zbase + kk * 64, 64), :], g0)
            pltpu.sync_copy(g0, out_h.at[c, pl.ds(zbase + kk * 64, 64), :])

    return passB


def kernel(x, edge_index, edge_weight, trans_W, eta, sws, swn, swa, swe, beta, sgW1, sgb1, sgW2, sgb2, s_ws, s_wn, s_wst, s_wa, s_we, tau, tgW1, tgb1, tgW2, tgb2, t_ws, t_wn, t_wst, t_wa, t_we, theta1, theta2, theta3, theta4):
    n = x.shape[0]
    e = edge_index.shape[1]
    NPAD = ((n + 1 + 255) // 256) * 256
    EP = ((e + 65535) // 65536) * 65536
    G = EP // 128
    NB = 10
    BR = n // NB

    src = edge_index[0]
    dst = edge_index[1]
    seed = x[:, -1]
    source = x[:, :H]
    target = x[:, H:2 * H]

    pad_i = jnp.full((EP - e,), n, i32)
    srcR = jnp.concatenate([src, pad_i]).reshape(G, 128)
    dstR = jnp.concatenate([dst, pad_i]).reshape(G, 128)
    ewR = jnp.concatenate([edge_weight, jnp.zeros((EP - e,), f32)]).reshape(G, 128)

    passA = _make_passA(NPAD, G)
    passW = _make_passW(NPAD, G)
    passB = _make_passB(NPAD, G)

    k1 = pl.pallas_call(
        _k1_body,
        out_shape=(jax.ShapeDtypeStruct((n, H), f32),) * 4
        + (jax.ShapeDtypeStruct((n, 8), f32),),
        grid=(NB,),
        in_specs=[
            pl.BlockSpec((BR, H), lambda i: (i, 0)),
            pl.BlockSpec((BR, H), lambda i: (i, 0)),
            pl.BlockSpec((BR, 1), lambda i: (i, 0)),
        ] + [pl.BlockSpec((H, H), lambda i: (0, 0)),
             pl.BlockSpec((H, 8), lambda i: (0, 0)),
             pl.BlockSpec((H, 8), lambda i: (0, 0)),
             pl.BlockSpec((H, H), lambda i: (0, 0)),
             pl.BlockSpec((1, H), lambda i: (0, 0)),
             pl.BlockSpec((H, H), lambda i: (0, 0)),
             pl.BlockSpec((1, H), lambda i: (0, 0)),
             pl.BlockSpec((H, H), lambda i: (0, 0)),
             pl.BlockSpec((1, H), lambda i: (0, 0)),
             pl.BlockSpec((H, H), lambda i: (0, 0)),
             pl.BlockSpec((1, H), lambda i: (0, 0))],
        out_specs=[pl.BlockSpec((BR, H), lambda i: (i, 0))] * 4
        + [pl.BlockSpec((BR, 8), lambda i: (i, 0))],
    )

    k2 = pl.pallas_call(
        _k2_body,
        out_shape=(jax.ShapeDtypeStruct((1, NPAD), f32),
                   jax.ShapeDtypeStruct((2, NPAD), f32)),
    )

    k3 = pl.pallas_call(
        _k3_body,
        out_shape=(jax.ShapeDtypeStruct((n, H), f32),) * 2,
        grid=(NB,),
        in_specs=[pl.BlockSpec((BR, H), lambda i: (i, 0))] * 4
        + [pl.BlockSpec((BR, 1), lambda i: (i, 0)),
           pl.BlockSpec((1, 8), lambda i: (0, 0))],
        out_specs=[pl.BlockSpec((BR, H), lambda i: (i, 0))] * 2,
    )

    kf = pl.pallas_call(
        _kf_body,
        out_shape=jax.ShapeDtypeStruct((n, 1), f32),
        grid=(NB,),
        in_specs=[
            pl.BlockSpec((BR, H), lambda i: (i, 0)),
            pl.BlockSpec((BR, H), lambda i: (i, 0)),
            pl.BlockSpec((BR, 1), lambda i: (i, 0)),
        ] + [pl.BlockSpec((H, H), lambda i: (0, 0)),
             pl.BlockSpec((H, H), lambda i: (0, 0)),
             pl.BlockSpec((H, H), lambda i: (0, 0)),
             pl.BlockSpec((3 * H, 1), lambda i: (0, 0))],
        out_specs=pl.BlockSpec((BR, 1), lambda i: (i, 0)),
    )

    state = seed
    zrow = jnp.zeros((NPAD - n, H), f32)
    seed_p = jnp.concatenate([seed, jnp.zeros((NPAD - n,), f32)]).reshape(1, NPAD)

    for i in range(T):
        a8 = jnp.concatenate(
            [eta[i][:H], beta[i][:H], tau[i][:H], jnp.zeros((H, 5), f32)], axis=1)
        b8 = jnp.concatenate(
            [jnp.zeros((H, 3), f32), eta[i][H:], beta[i][H:], tau[i][H:],
             jnp.zeros((H, 2), f32)], axis=1)
        tsi, tti, sgate, tgate, proj = k1(
            source, target, state[:, None], trans_W[i], a8, b8,
            sgW1[i].T, sgb1[i][None, :], sgW2[i].T, sgb2[i][None, :],
            tgW1[i].T, tgb1[i][None, :], tgW2[i].T, tgb2[i][None, :])

        ns = jnp.concatenate([proj, jnp.zeros((NPAD - n, 8), f32)], axis=0).reshape(NPAD * 8)
        sums, expfgR = passA(srcR, dstR, ewR, ns)
        sums = sums.reshape(2, 5, NPAD)

        par2 = jnp.stack([sws[i], swn[i], swa[i], swe[i],
                          jnp.zeros((), f32), jnp.zeros((), f32),
                          jnp.zeros((), f32), jnp.zeros((), f32)])[None, :]
        state_p = jnp.concatenate([state, jnp.zeros((NPAD - n,), f32)]).reshape(1, NPAD)
        nst, sinv = k2(sums, state_p, seed_p, par2)

        wa2 = jnp.concatenate([jnp.full((16,), s_wa[i], f32), jnp.full((16,), t_wa[i], f32)])
        wb2 = jnp.concatenate([jnp.full((16,), s_we[i], f32), jnp.full((16,), t_we[i], f32)])
        gidxM = jnp.concatenate([dstR, srcR + NPAD], axis=0).reshape(-1, 64)
        sidxM = jnp.concatenate([srcR, dstR], axis=0).reshape(-1, 64)
        gate32 = jnp.concatenate([sgate, zrow, tgate, zrow], axis=0)
        alph = passW(sidxM, ewR, expfgR, sinv.reshape(2 * NPAD), wa2, wb2)
        bc = passB(gidxM, sidxM, alph, gate32)

        new_state = nst[0, :n]
        par3 = jnp.stack([s_ws[i], s_wn[i], s_wst[i],
                          t_ws[i], t_wn[i], t_wst[i],
                          jnp.zeros((), f32), jnp.zeros((), f32)])[None, :]
        source, target = k3(tsi, tti, bc[0, :n], bc[1, :n],
                            new_state[:, None], par3)
        state = new_state

    q = kf(source, target, state[:, None], theta2, theta3, theta4, theta1)
    return q[:, 0]


# speedup vs baseline: 1.0239x; 1.0193x over previous
import functools

import jax
import jax.numpy as jnp
from jax import lax
from jax.experimental import pallas as pl
from jax.experimental.pallas import tpu as pltpu
from jax.experimental.pallas import tpu_sc as plsc

H = 128
T = 2
f32 = jnp.float32
i32 = jnp.int32


def _lrelu(v):
    return jnp.where(v > 0, v, 0.2 * v)


def _k1_body(src_ref, tgt_ref, st_ref, w_ref, a8_ref, b8_ref,
             sg1_ref, sgb1_ref, sg2_ref, sgb2_ref,
             tg1_ref, tgb1_ref, tg2_ref, tgb2_ref,
             tsi_ref, tti_ref, sgate_ref, tgate_ref, proj_ref):
    s = src_ref[...]
    t = tgt_ref[...]
    w = w_ref[...]
    tsi = jnp.dot(s, w, preferred_element_type=f32)
    tti = jnp.dot(t, w, preferred_element_type=f32)
    tsi_ref[...] = tsi
    tti_ref[...] = tti
    proj = (jnp.dot(tsi, a8_ref[...], preferred_element_type=f32)
            + jnp.dot(tti, b8_ref[...], preferred_element_type=f32))
    col = lax.broadcasted_iota(i32, (1, 8), 1)
    proj_ref[...] = jnp.where(col == 6, st_ref[...], proj)
    h1 = _lrelu(jnp.dot(t, sg1_ref[...], preferred_element_type=f32) + sgb1_ref[...])
    sgate_ref[...] = _lrelu(jnp.dot(h1, sg2_ref[...], preferred_element_type=f32) + sgb2_ref[...])
    h2 = _lrelu(jnp.dot(s, tg1_ref[...], preferred_element_type=f32) + tgb1_ref[...])
    tgate_ref[...] = _lrelu(jnp.dot(h2, tg2_ref[...], preferred_element_type=f32) + tgb2_ref[...])


def _k2_body(sums_ref, st_ref, seed_ref, par_ref, nst_ref, sinv_ref):
    smat = sums_ref[0] + sums_ref[1]
    sE = smat[0:1]
    sES = smat[1:2]
    sEW = smat[2:3]
    sFG = smat[3:5]
    sws = par_ref[:, 0:1]
    swn = par_ref[:, 1:2]
    swa = par_ref[:, 2:3]
    swe = par_ref[:, 3:4]
    a_v = swa * sES / (sE + 1e-16) + swe * sEW
    st = st_ref[...]
    seed = seed_ref[...]
    nst = jax.nn.sigmoid(st * sws + a_v * swn)
    nst_ref[...] = nst * (1.0 - seed) + seed
    sinv_ref[...] = 1.0 / (sFG + 1e-16)


def _k3_body(tsi_ref, tti_ref, bv_ref, cv_ref, nst_ref, par_ref, ns_ref, nt_ref):
    p = par_ref
    nst = nst_ref[...]
    bv = bv_ref[...].astype(f32)
    cv = cv_ref[...].astype(f32)
    ns_ref[...] = _lrelu(tsi_ref[...] * p[:, 0:1] + bv * p[:, 1:2] + nst * p[:, 2:3])
    nt_ref[...] = _lrelu(tti_ref[...] * p[:, 3:4] + cv * p[:, 4:5] + nst * p[:, 5:6])


def _kf_body(src_ref, tgt_ref, st_ref, th2_ref, th3_ref, th4_ref, th1_ref, o_ref):
    s = src_ref[...]
    t = tgt_ref[...]
    st = st_ref[...]
    f1 = _lrelu(jnp.dot(s, th2_ref[...], preferred_element_type=f32))
    f2 = _lrelu(jnp.dot(t, th3_ref[...], preferred_element_type=f32))
    f3 = _lrelu(jnp.dot(st * s, th4_ref[...], preferred_element_type=f32))
    th1 = th1_ref[...]
    o_ref[...] = (jnp.dot(f1, th1[:H], preferred_element_type=f32)
                  + jnp.dot(f2, th1[H:2 * H], preferred_element_type=f32)
                  + jnp.dot(f3, th1[2 * H:], preferred_element_type=f32))


def _k1f_body(tsi_ref, tti_ref, bv_ref, cv_ref, nst_ref, p3_ref,
              w_ref, a8_ref, b8_ref,
              sg1_ref, sgb1_ref, sg2_ref, sgb2_ref,
              tg1_ref, tgb1_ref, tg2_ref, tgb2_ref,
              tso_ref, tto_ref, sgate_ref, tgate_ref, proj_ref):
    p = p3_ref
    nst = nst_ref[...]
    s = _lrelu(tsi_ref[...] * p[:, 0:1] + bv_ref[...] * p[:, 1:2] + nst * p[:, 2:3])
    t = _lrelu(tti_ref[...] * p[:, 3:4] + cv_ref[...] * p[:, 4:5] + nst * p[:, 5:6])
    w = w_ref[...]
    tsi = jnp.dot(s, w, preferred_element_type=f32)
    tti = jnp.dot(t, w, preferred_element_type=f32)
    tso_ref[...] = tsi
    tto_ref[...] = tti
    proj = (jnp.dot(tsi, a8_ref[...], preferred_element_type=f32)
            + jnp.dot(tti, b8_ref[...], preferred_element_type=f32))
    col = lax.broadcasted_iota(i32, (1, 8), 1)
    proj_ref[...] = jnp.where(col == 6, nst, proj)
    h1 = _lrelu(jnp.dot(t, sg1_ref[...], preferred_element_type=f32) + sgb1_ref[...])
    sgate_ref[...] = _lrelu(jnp.dot(h1, sg2_ref[...], preferred_element_type=f32) + sgb2_ref[...])
    h2 = _lrelu(jnp.dot(s, tg1_ref[...], preferred_element_type=f32) + tgb1_ref[...])
    tgate_ref[...] = _lrelu(jnp.dot(h2, tg2_ref[...], preferred_element_type=f32) + tgb2_ref[...])


def _kff_body(tsi_ref, tti_ref, bv_ref, cv_ref, nst_ref, p3_ref,
              th2_ref, th3_ref, th4_ref, th1_ref, o_ref):
    p = p3_ref
    nst = nst_ref[...]
    s = _lrelu(tsi_ref[...] * p[:, 0:1] + bv_ref[...] * p[:, 1:2] + nst * p[:, 2:3])
    t = _lrelu(tti_ref[...] * p[:, 3:4] + cv_ref[...] * p[:, 4:5] + nst * p[:, 5:6])
    f1 = _lrelu(jnp.dot(s, th2_ref[...], preferred_element_type=f32))
    f2 = _lrelu(jnp.dot(t, th3_ref[...], preferred_element_type=f32))
    f3 = _lrelu(jnp.dot(nst * s, th4_ref[...], preferred_element_type=f32))
    th1 = th1_ref[...]
    o_ref[...] = (jnp.dot(f1, th1[:H], preferred_element_type=f32)
                  + jnp.dot(f2, th1[H:2 * H], preferred_element_type=f32)
                  + jnp.dot(f3, th1[2 * H:], preferred_element_type=f32))


def _make_passA(NPAD, G):
    RA = G // 32
    CHA = RA // 16
    mesh = plsc.VectorSubcoreMesh(core_axis_name="c", subcore_axis_name="s")

    @functools.partial(
        pl.kernel,
        out_type=(jax.ShapeDtypeStruct((2 * 5 * NPAD,), f32),
                  jax.ShapeDtypeStruct((2 * G, 128), f32)),
        mesh=mesh,
        compiler_params=pltpu.CompilerParams(needs_layout_passes=False),
        scratch_types=[
            pltpu.VMEM((NPAD * 8,), f32),
            pltpu.VMEM((16, 128), i32),
            pltpu.VMEM((16, 128), i32),
            pltpu.VMEM((16, 128), f32),
            pltpu.VMEM((16, 128), f32),
            pltpu.VMEM((16, 128), f32),
            pltpu.VMEM((16, 128), f32),
            pltpu.VMEM((16, 128), f32),
            pltpu.VMEM((16, 128), f32),
            pltpu.VMEM((NPAD,), f32),
            pltpu.VMEM_SHARED((NPAD,), f32),
            pltpu.VMEM_SHARED((NPAD,), f32),
            pltpu.VMEM_SHARED((NPAD,), f32),
            pltpu.VMEM_SHARED((NPAD,), f32),
            pltpu.VMEM_SHARED((NPAD,), f32),
            pltpu.SemaphoreType.DMA,
        ],
    )
    def passA(src_h, dst_h, ew_h, ns_h, sums_h, expfg_h,
              ns_v, src2, dst2, ew2, vale, vales, valew, valf, valg, stage,
              acc_e, acc_es, acc_ew, acc_f, acc_g, sem):
        c = lax.axis_index("c")
        s = lax.axis_index("s")
        accs = [acc_e, acc_es, acc_ew, acc_f, acc_g]

        def zb(k, carry):
            stage[pl.ds(k * 16, 16)] = jnp.zeros((16,), f32)
            return carry
        lax.fori_loop(0, NPAD // 16, zb, 0)
        for ai in range(5):
            @pl.when(s == ai)
            def _(ai=ai):
                pltpu.sync_copy(stage, accs[ai])
        plsc.subcore_barrier()

        pltpu.sync_copy(ns_h, ns_v)
        base = (c * 16 + s) * RA

        def chunk(ch, carry):
            rb = base + ch * 16
            pltpu.sync_copy(src_h.at[pl.ds(rb, 16), :], src2)
            pltpu.sync_copy(dst_h.at[pl.ds(rb, 16), :], dst2)
            pltpu.sync_copy(ew_h.at[pl.ds(rb, 16), :], ew2)

            def row(i, c2):
                for k in range(8):
                    sl = pl.ds(k * 16, 16)
                    si = src2[i, sl] * 8
                    di = dst2[i, sl] * 8
                    w = ew2[i, sl]
                    es = plsc.load_gather(ns_v, [si])
                    fs = plsc.load_gather(ns_v, [si + 1])
                    gs = plsc.load_gather(ns_v, [si + 2])
                    et = plsc.load_gather(ns_v, [di + 3])
                    ft = plsc.load_gather(ns_v, [di + 4])
                    gt = plsc.load_gather(ns_v, [di + 5])
                    st = plsc.load_gather(ns_v, [si + 6])
                    e = es + et
                    expe = jnp.exp(jnp.where(e > 0, e, 0.2 * e))
                    f = fs + ft
                    expf = jnp.exp(jnp.where(f > 0, f, 0.2 * f))
                    g = gs + gt
                    expg = jnp.exp(jnp.where(g > 0, g, 0.2 * g))
                    vale[i, sl] = expe
                    vales[i, sl] = expe * st
                    valew[i, sl] = w * st
                    valf[i, sl] = expf
                    valg[i, sl] = expg
                return c2
            lax.fori_loop(0, 16, row, 0)

            pltpu.sync_copy(valf, expfg_h.at[pl.ds(rb, 16), :])
            pltpu.sync_copy(valg, expfg_h.at[pl.ds(G + rb, 16), :])
            pend = []
            for i in range(16):
                pend.append([
                    pltpu.async_copy(vale.at[i], acc_e.at[dst2.at[i]], sem, add=True),
                    pltpu.async_copy(vales.at[i], acc_es.at[dst2.at[i]], sem, add=True),
                    pltpu.async_copy(valew.at[i], acc_ew.at[dst2.at[i]], sem, add=True),
                    pltpu.async_copy(valf.at[i], acc_f.at[src2.at[i]], sem, add=True),
                    pltpu.async_copy(valg.at[i], acc_g.at[dst2.at[i]], sem, add=True),
                ])
                if i >= 2:
                    for dd in pend[i - 2]:
                        dd.wait()
            for row_d in pend[14:]:
                for dd in row_d:
                    dd.wait()
            return carry
        lax.fori_loop(0, CHA, chunk, 0)
        plsc.subcore_barrier()
        for ai in range(5):
            @pl.when(s == ai)
            def _(ai=ai):
                pltpu.sync_copy(accs[ai], stage)
                pltpu.sync_copy(stage, sums_h.at[pl.ds((c * 5 + ai) * NPAD, NPAD)])

    return passA


def _make_passW(NPAD, G):
    RB = G // 16
    CHB = RB // 16
    mesh = plsc.VectorSubcoreMesh(core_axis_name="c", subcore_axis_name="s")

    @functools.partial(
        pl.kernel,
        out_type=jax.ShapeDtypeStruct((2 * G * 128,), f32),
        mesh=mesh,
        compiler_params=pltpu.CompilerParams(needs_layout_passes=False),
        scratch_types=[
            pltpu.VMEM((NPAD,), f32),
            pltpu.VMEM((32, 64), i32),
            pltpu.VMEM((16, 128), f32),
            pltpu.VMEM((16, 128), f32),
            pltpu.VMEM((2048,), f32),
            pltpu.VMEM((16,), f32),
            pltpu.VMEM((16,), f32),
        ],
    )
    def passW(sidx_h, ew_h, exp_h, sinv_h, wa_h, wb_h, alph_h,
              sinv_v, sidx2, ew2, exp2, alpha_f, wa_v, wb_v):
        c = lax.axis_index("c")
        s = lax.axis_index("s")
        pltpu.sync_copy(wa_h.at[pl.ds(c * 16, 16)], wa_v)
        pltpu.sync_copy(wb_h.at[pl.ds(c * 16, 16)], wb_v)
        pltpu.sync_copy(sinv_h.at[pl.ds(c * NPAD, NPAD)], sinv_v)
        wa = wa_v[...]
        wb = wb_v[...]
        base = s * RB
        cbase = c * G + base

        def chunk0(ch, carry):
            pltpu.sync_copy(ew_h.at[pl.ds(base + ch * 16, 16), :], ew2)
            pltpu.sync_copy(exp_h.at[pl.ds(cbase + ch * 16, 16), :], exp2)
            pltpu.sync_copy(sidx_h.at[pl.ds((cbase + ch * 16) * 2, 32), :],
                            sidx2)

            def arow(i, c2):
                for k in range(8):
                    sl = pl.ds(k * 16, 16)
                    sg = plsc.load_gather(
                        sinv_v, [sidx2[2 * i + k // 4, pl.ds(16 * (k % 4), 16)]])
                    alpha_f[pl.ds(i * 128 + k * 16, 16)] = (
                        exp2[i, sl] * sg * wa + ew2[i, sl] * wb)
                return c2
            lax.fori_loop(0, 16, arow, 0)
            pltpu.sync_copy(alpha_f,
                            alph_h.at[pl.ds((cbase + ch * 16) * 128, 2048)])
            return carry
        lax.fori_loop(0, CHB, chunk0, 0)

    return passW

def _make_passB(NPAD, G):
    RB = G // 16
    CHB = RB // 16
    ZR = NPAD // 16
    ZFULL, ZREM = ZR // 128, ZR % 128
    mesh = plsc.VectorSubcoreMesh(core_axis_name="c", subcore_axis_name="s")

    @functools.partial(
        pl.kernel,
        out_type=jax.ShapeDtypeStruct((2, NPAD, H), f32),
        mesh=mesh,
        compiler_params=pltpu.CompilerParams(needs_layout_passes=False),
        scratch_types=[
            pltpu.VMEM((32, 64), i32),
            pltpu.VMEM((32, 64), i32),
            pltpu.VMEM((2048,), f32),
            pltpu.VMEM((64, H), f32),
            pltpu.VMEM((64, H), f32),
            pltpu.VMEM((64, H), f32),
            pltpu.VMEM((64, H), f32),
            pltpu.VMEM_SHARED((NPAD, H), f32),
            pltpu.SemaphoreType.DMA,
            pltpu.SemaphoreType.DMA,
            pltpu.SemaphoreType.DMA,
            pltpu.SemaphoreType.DMA,
            pltpu.SemaphoreType.DMA,
            pltpu.SemaphoreType.DMA,
            pltpu.SemaphoreType.DMA,
            pltpu.SemaphoreType.DMA,
        ],
    )
    def passB(gidx_h, sidx_h, alph_h, gate_h, out_h,
              gidx2, sidx2, alpha_f, g0, g1, g2, g3, acc,
              sg0, sg1, sg2, sg3, ss0, ss1, ss2, ss3):
        c = lax.axis_index("c")
        s = lax.axis_index("s")
        gbuf = [g0, g1, g2, g3]
        bbuf = gbuf
        sem_g = [sg0, sg1, sg2, sg3]
        sem_s = [ss0, ss1, ss2, ss3]
        ZR = NPAD // 16
        Z64 = ZR // 64

        def zr(k, carry):
            for v in range(8):
                g0[k, pl.ds(v * 16, 16)] = jnp.zeros((16,), f32)
            return carry
        lax.fori_loop(0, 64, zr, 0)
        zbase = s * ZR
        for kk in range(Z64):
            pltpu.sync_copy(g0, acc.at[pl.ds(zbase + kk * 64, 64), :])
        plsc.subcore_barrier()

        base = s * RB
        cbase = c * G + base

        bidx = [jnp.full((16,), q, i32) for q in range(16)]

        def scale(gb, bb, t):
            def grp(g, carry):
                v = alpha_f[pl.ds(t * 64 + g * 16, 16)]
                for q in range(16):
                    av = v.at[bidx[q]].get(mode="promise_in_bounds")
                    r = g * 16 + q
                    for vv in range(8):
                        slv = pl.ds(vv * 16, 16)
                        gb[r, slv] = gb[r, slv] * av
                return carry
            lax.fori_loop(0, 4, grp, 0)

        def gfire(t, b):
            pltpu.async_copy(gate_h.at[gidx2.at[t]], gbuf[b], sem_g[b])

        def gwait(t, b):
            pltpu.make_async_copy(
                gate_h.at[gidx2.at[t]], gbuf[b], sem_g[b]).wait()

        def sfire(t, b):
            pltpu.async_copy(bbuf[b], acc.at[sidx2.at[t]], sem_s[b],
                             add=True)

        def swait(b):
            pltpu.make_async_copy(
                bbuf[b], acc.at[sidx2.at[0]], sem_s[b]).wait()

        def chunk1(ch, carry):
            hb2 = (cbase + ch * 16) * 2
            pltpu.sync_copy(gidx_h.at[pl.ds(hb2, 32), :], gidx2)
            pltpu.sync_copy(sidx_h.at[pl.ds(hb2, 32), :], sidx2)
            pltpu.sync_copy(alph_h.at[pl.ds((cbase + ch * 16) * 128, 2048)],
                            alpha_f)
            gfire(0, 0)
            gfire(1, 1)

            def quad(tq, c2):
                t0 = tq * 4
                for b in range(4):
                    t = t0 + b

                    @pl.when((t >= 2) | (ch > 0))
                    def _():
                        swait((b + 2) % 4)

                    @pl.when(t + 2 < 32)
                    def _():
                        gfire(t + 2, (b + 2) % 4)
                    gwait(t, b)
                    scale(gbuf[b], bbuf[b], t)
                    sfire(t, b)
                return c2
            lax.fori_loop(0, 8, quad, 0)
            return carry
        lax.fori_loop(0, CHB, chunk1, 0)
        swait(2)
        swait(3)

        plsc.subcore_barrier()
        for kk in range(Z64):
            pltpu.sync_copy(acc.at[pl.ds(zbase + kk * 64, 64), :], g0)
            pltpu.sync_copy(g0, out_h.at[c, pl.ds(zbase + kk * 64, 64), :])

    return passB


def kernel(x, edge_index, edge_weight, trans_W, eta, sws, swn, swa, swe, beta, sgW1, sgb1, sgW2, sgb2, s_ws, s_wn, s_wst, s_wa, s_we, tau, tgW1, tgb1, tgW2, tgb2, t_ws, t_wn, t_wst, t_wa, t_we, theta1, theta2, theta3, theta4):
    n = x.shape[0]
    e = edge_index.shape[1]
    NPAD = ((n + 1 + 255) // 256) * 256
    EP = ((e + 65535) // 65536) * 65536
    G = EP // 128
    NB = 10
    BR = n // NB

    src = edge_index[0]
    dst = edge_index[1]
    seed = x[:, -1]
    source = x[:, :H]
    target = x[:, H:2 * H]

    pad_i = jnp.full((EP - e,), n, i32)
    srcR = jnp.concatenate([src, pad_i]).reshape(G, 128)
    dstR = jnp.concatenate([dst, pad_i]).reshape(G, 128)
    ewR = jnp.concatenate([edge_weight, jnp.zeros((EP - e,), f32)]).reshape(G, 128)

    passA = _make_passA(NPAD, G)
    passW = _make_passW(NPAD, G)
    passB = _make_passB(NPAD, G)

    k1 = pl.pallas_call(
        _k1_body,
        out_shape=(jax.ShapeDtypeStruct((n, H), f32),) * 4
        + (jax.ShapeDtypeStruct((n, 8), f32),),
        grid=(NB,),
        in_specs=[
            pl.BlockSpec((BR, H), lambda i: (i, 0)),
            pl.BlockSpec((BR, H), lambda i: (i, 0)),
            pl.BlockSpec((BR, 1), lambda i: (i, 0)),
        ] + [pl.BlockSpec((H, H), lambda i: (0, 0)),
             pl.BlockSpec((H, 8), lambda i: (0, 0)),
             pl.BlockSpec((H, 8), lambda i: (0, 0)),
             pl.BlockSpec((H, H), lambda i: (0, 0)),
             pl.BlockSpec((1, H), lambda i: (0, 0)),
             pl.BlockSpec((H, H), lambda i: (0, 0)),
             pl.BlockSpec((1, H), lambda i: (0, 0)),
             pl.BlockSpec((H, H), lambda i: (0, 0)),
             pl.BlockSpec((1, H), lambda i: (0, 0)),
             pl.BlockSpec((H, H), lambda i: (0, 0)),
             pl.BlockSpec((1, H), lambda i: (0, 0))],
        out_specs=[pl.BlockSpec((BR, H), lambda i: (i, 0))] * 4
        + [pl.BlockSpec((BR, 8), lambda i: (i, 0))],
    )

    k2 = pl.pallas_call(
        _k2_body,
        out_shape=(jax.ShapeDtypeStruct((1, NPAD), f32),
                   jax.ShapeDtypeStruct((2, NPAD), f32)),
    )

    _pre_specs = [pl.BlockSpec((BR, H), lambda i: (i, 0))] * 4 + [
        pl.BlockSpec((BR, 1), lambda i: (i, 0)),
        pl.BlockSpec((1, 8), lambda i: (0, 0))]

    k1f = pl.pallas_call(
        _k1f_body,
        out_shape=(jax.ShapeDtypeStruct((n, H), f32),) * 4
        + (jax.ShapeDtypeStruct((n, 8), f32),),
        grid=(NB,),
        in_specs=_pre_specs
        + [pl.BlockSpec((H, H), lambda i: (0, 0)),
           pl.BlockSpec((H, 8), lambda i: (0, 0)),
           pl.BlockSpec((H, 8), lambda i: (0, 0)),
           pl.BlockSpec((H, H), lambda i: (0, 0)),
           pl.BlockSpec((1, H), lambda i: (0, 0)),
           pl.BlockSpec((H, H), lambda i: (0, 0)),
           pl.BlockSpec((1, H), lambda i: (0, 0)),
           pl.BlockSpec((H, H), lambda i: (0, 0)),
           pl.BlockSpec((1, H), lambda i: (0, 0)),
           pl.BlockSpec((H, H), lambda i: (0, 0)),
           pl.BlockSpec((1, H), lambda i: (0, 0))],
        out_specs=[pl.BlockSpec((BR, H), lambda i: (i, 0))] * 4
        + [pl.BlockSpec((BR, 8), lambda i: (i, 0))],
    )

    kff = pl.pallas_call(
        _kff_body,
        out_shape=jax.ShapeDtypeStruct((n, 1), f32),
        grid=(NB,),
        in_specs=_pre_specs
        + [pl.BlockSpec((H, H), lambda i: (0, 0)),
           pl.BlockSpec((H, H), lambda i: (0, 0)),
           pl.BlockSpec((H, H), lambda i: (0, 0)),
           pl.BlockSpec((3 * H, 1), lambda i: (0, 0))],
        out_specs=pl.BlockSpec((BR, 1), lambda i: (i, 0)),
    )

    zrow = jnp.zeros((NPAD - n, H), f32)
    seed_p = jnp.concatenate([seed, jnp.zeros((NPAD - n,), f32)]).reshape(1, NPAD)
    gidxM = jnp.concatenate([dstR, srcR + NPAD], axis=0).reshape(-1, 64)
    sidxM = jnp.concatenate([srcR, dstR], axis=0).reshape(-1, 64)

    def edge_phase(i, proj, sgate, tgate, state_p):
        ns = jnp.concatenate([proj, jnp.zeros((NPAD - n, 8), f32)], axis=0).reshape(NPAD * 8)
        sums, expfgR = passA(srcR, dstR, ewR, ns)
        par2 = jnp.stack([sws[i], swn[i], swa[i], swe[i],
                          jnp.zeros((), f32), jnp.zeros((), f32),
                          jnp.zeros((), f32), jnp.zeros((), f32)])[None, :]
        nst, sinv = k2(sums.reshape(2, 5, NPAD), state_p, seed_p, par2)
        wa2 = jnp.concatenate([jnp.full((16,), s_wa[i], f32),
                               jnp.full((16,), t_wa[i], f32)])
        wb2 = jnp.concatenate([jnp.full((16,), s_we[i], f32),
                               jnp.full((16,), t_we[i], f32)])
        gate32 = jnp.concatenate([sgate, zrow, tgate, zrow], axis=0)
        alph = passW(sidxM, ewR, expfgR, sinv.reshape(2 * NPAD), wa2, wb2)
        bc = passB(gidxM, sidxM, alph, gate32)
        par3 = jnp.stack([s_ws[i], s_wn[i], s_wst[i],
                          t_ws[i], t_wn[i], t_wst[i],
                          jnp.zeros((), f32), jnp.zeros((), f32)])[None, :]
        return nst, bc, par3

    def weights(i):
        a8 = jnp.concatenate(
            [eta[i][:H], beta[i][:H], tau[i][:H], jnp.zeros((H, 5), f32)], axis=1)
        b8 = jnp.concatenate(
            [jnp.zeros((H, 3), f32), eta[i][H:], beta[i][H:], tau[i][H:],
             jnp.zeros((H, 2), f32)], axis=1)
        return [trans_W[i], a8, b8,
                sgW1[i].T, sgb1[i][None, :], sgW2[i].T, sgb2[i][None, :],
                tgW1[i].T, tgb1[i][None, :], tgW2[i].T, tgb2[i][None, :]]

    tsi, tti, sgate, tgate, proj = k1(source, target, seed[:, None], *weights(0))
    nst, bc, par3 = edge_phase(0, proj, sgate, tgate, seed_p)

    tsi, tti, sgate, tgate, proj = k1f(
        tsi, tti, bc[0, :n], bc[1, :n], nst[0, :n, None], par3, *weights(1))
    nst, bc, par3 = edge_phase(1, proj, sgate, tgate, nst[:, :NPAD])

    q = kff(tsi, tti, bc[0, :n], bc[1, :n], nst[0, :n, None], par3,
            theta2, theta3, theta4, theta1)
    return q[:, 0]


# speedup vs baseline: 1.0244x; 1.0004x over previous
import functools

import jax
import jax.numpy as jnp
from jax import lax
from jax.experimental import pallas as pl
from jax.experimental.pallas import tpu as pltpu
from jax.experimental.pallas import tpu_sc as plsc

H = 128
T = 2
f32 = jnp.float32
i32 = jnp.int32


def _lrelu(v):
    return jnp.where(v > 0, v, 0.2 * v)


def _k1_body(src_ref, tgt_ref, st_ref, w_ref, a8_ref, b8_ref,
             sg1_ref, sgb1_ref, sg2_ref, sgb2_ref,
             tg1_ref, tgb1_ref, tg2_ref, tgb2_ref,
             tsi_ref, tti_ref, sgate_ref, tgate_ref, proj_ref):
    s = src_ref[...]
    t = tgt_ref[...]
    w = w_ref[...]
    tsi = jnp.dot(s, w, preferred_element_type=f32)
    tti = jnp.dot(t, w, preferred_element_type=f32)
    tsi_ref[...] = tsi
    tti_ref[...] = tti
    proj = (jnp.dot(tsi, a8_ref[...], preferred_element_type=f32)
            + jnp.dot(tti, b8_ref[...], preferred_element_type=f32))
    col = lax.broadcasted_iota(i32, (1, 8), 1)
    proj_ref[...] = jnp.where(col == 6, st_ref[...], proj)
    h1 = _lrelu(jnp.dot(t, sg1_ref[...], preferred_element_type=f32) + sgb1_ref[...])
    sgate_ref[...] = _lrelu(jnp.dot(h1, sg2_ref[...], preferred_element_type=f32) + sgb2_ref[...])
    h2 = _lrelu(jnp.dot(s, tg1_ref[...], preferred_element_type=f32) + tgb1_ref[...])
    tgate_ref[...] = _lrelu(jnp.dot(h2, tg2_ref[...], preferred_element_type=f32) + tgb2_ref[...])


def _k2_body(sums_ref, st_ref, seed_ref, par_ref, nst_ref, sinv_ref):
    smat = sums_ref[0] + sums_ref[1]
    sE = smat[0:1]
    sES = smat[1:2]
    sEW = smat[2:3]
    sFG = smat[3:5]
    sws = par_ref[:, 0:1]
    swn = par_ref[:, 1:2]
    swa = par_ref[:, 2:3]
    swe = par_ref[:, 3:4]
    a_v = swa * sES / (sE + 1e-16) + swe * sEW
    st = st_ref[...]
    seed = seed_ref[...]
    nst = jax.nn.sigmoid(st * sws + a_v * swn)
    nst_ref[...] = nst * (1.0 - seed) + seed
    sinv_ref[...] = 1.0 / (sFG + 1e-16)


def _k1f_body(tsi_ref, tti_ref, bv_ref, cv_ref, nst_ref, p3_ref,
              w_ref, a8_ref, b8_ref,
              sg1_ref, sgb1_ref, sg2_ref, sgb2_ref,
              tg1_ref, tgb1_ref, tg2_ref, tgb2_ref,
              tso_ref, tto_ref, sgate_ref, tgate_ref, proj_ref):
    p = p3_ref
    nst = nst_ref[...]
    s = _lrelu(tsi_ref[...] * p[:, 0:1] + bv_ref[...] * p[:, 1:2] + nst * p[:, 2:3])
    t = _lrelu(tti_ref[...] * p[:, 3:4] + cv_ref[...] * p[:, 4:5] + nst * p[:, 5:6])
    w = w_ref[...]
    tsi = jnp.dot(s, w, preferred_element_type=f32)
    tti = jnp.dot(t, w, preferred_element_type=f32)
    tso_ref[...] = tsi
    tto_ref[...] = tti
    proj = (jnp.dot(tsi, a8_ref[...], preferred_element_type=f32)
            + jnp.dot(tti, b8_ref[...], preferred_element_type=f32))
    col = lax.broadcasted_iota(i32, (1, 8), 1)
    proj_ref[...] = jnp.where(col == 6, nst, proj)
    h1 = _lrelu(jnp.dot(t, sg1_ref[...], preferred_element_type=f32) + sgb1_ref[...])
    sgate_ref[...] = _lrelu(jnp.dot(h1, sg2_ref[...], preferred_element_type=f32) + sgb2_ref[...])
    h2 = _lrelu(jnp.dot(s, tg1_ref[...], preferred_element_type=f32) + tgb1_ref[...])
    tgate_ref[...] = _lrelu(jnp.dot(h2, tg2_ref[...], preferred_element_type=f32) + tgb2_ref[...])


def _kff_body(tsi_ref, tti_ref, bv_ref, cv_ref, nst_ref, p3_ref,
              th2_ref, th3_ref, th4_ref, th1_ref, o_ref):
    p = p3_ref
    nst = nst_ref[...]
    s = _lrelu(tsi_ref[...] * p[:, 0:1] + bv_ref[...] * p[:, 1:2] + nst * p[:, 2:3])
    t = _lrelu(tti_ref[...] * p[:, 3:4] + cv_ref[...] * p[:, 4:5] + nst * p[:, 5:6])
    f1 = _lrelu(jnp.dot(s, th2_ref[...], preferred_element_type=f32))
    f2 = _lrelu(jnp.dot(t, th3_ref[...], preferred_element_type=f32))
    f3 = _lrelu(jnp.dot(nst * s, th4_ref[...], preferred_element_type=f32))
    th1 = th1_ref[...]
    o_ref[...] = (jnp.dot(f1, th1[:H], preferred_element_type=f32)
                  + jnp.dot(f2, th1[H:2 * H], preferred_element_type=f32)
                  + jnp.dot(f3, th1[2 * H:], preferred_element_type=f32))


def _make_passA(NPAD, G):
    RA = G // 32
    CHA = RA // 16
    mesh = plsc.VectorSubcoreMesh(core_axis_name="c", subcore_axis_name="s")

    @functools.partial(
        pl.kernel,
        out_type=(jax.ShapeDtypeStruct((2 * 5 * NPAD,), f32),
                  jax.ShapeDtypeStruct((2 * G, 128), f32)),
        mesh=mesh,
        compiler_params=pltpu.CompilerParams(needs_layout_passes=False),
        scratch_types=[
            pltpu.VMEM((NPAD * 8,), f32),
            pltpu.VMEM((16, 128), i32),
            pltpu.VMEM((16, 128), i32),
            pltpu.VMEM((16, 128), f32),
            pltpu.VMEM((16, 128), f32),
            pltpu.VMEM((16, 128), f32),
            pltpu.VMEM((16, 128), f32),
            pltpu.VMEM((16, 128), f32),
            pltpu.VMEM((16, 128), f32),
            pltpu.VMEM((NPAD,), f32),
            pltpu.VMEM_SHARED((NPAD,), f32),
            pltpu.VMEM_SHARED((NPAD,), f32),
            pltpu.VMEM_SHARED((NPAD,), f32),
            pltpu.VMEM_SHARED((NPAD,), f32),
            pltpu.VMEM_SHARED((NPAD,), f32),
            pltpu.SemaphoreType.DMA,
        ],
    )
    def passA(src_h, dst_h, ew_h, ns_h, sums_h, expfg_h,
              ns_v, src2, dst2, ew2, vale, vales, valew, valf, valg, stage,
              acc_e, acc_es, acc_ew, acc_f, acc_g, sem):
        c = lax.axis_index("c")
        s = lax.axis_index("s")
        accs = [acc_e, acc_es, acc_ew, acc_f, acc_g]

        def zb(k, carry):
            stage[pl.ds(k * 16, 16)] = jnp.zeros((16,), f32)
            return carry
        lax.fori_loop(0, NPAD // 16, zb, 0)
        for ai in range(5):
            @pl.when(s == ai)
            def _(ai=ai):
                pltpu.sync_copy(stage, accs[ai])
        plsc.subcore_barrier()

        pltpu.sync_copy(ns_h, ns_v)
        base = (c * 16 + s) * RA

        def chunk(ch, carry):
            rb = base + ch * 16
            pltpu.sync_copy(src_h.at[pl.ds(rb, 16), :], src2)
            pltpu.sync_copy(dst_h.at[pl.ds(rb, 16), :], dst2)
            pltpu.sync_copy(ew_h.at[pl.ds(rb, 16), :], ew2)

            def row(i, c2):
                for k in range(8):
                    sl = pl.ds(k * 16, 16)
                    si = src2[i, sl] * 8
                    di = dst2[i, sl] * 8
                    w = ew2[i, sl]
                    es = plsc.load_gather(ns_v, [si])
                    fs = plsc.load_gather(ns_v, [si + 1])
                    gs = plsc.load_gather(ns_v, [si + 2])
                    et = plsc.load_gather(ns_v, [di + 3])
                    ft = plsc.load_gather(ns_v, [di + 4])
                    gt = plsc.load_gather(ns_v, [di + 5])
                    st = plsc.load_gather(ns_v, [si + 6])
                    e = es + et
                    expe = jnp.exp(jnp.where(e > 0, e, 0.2 * e))
                    f = fs + ft
                    expf = jnp.exp(jnp.where(f > 0, f, 0.2 * f))
                    g = gs + gt
                    expg = jnp.exp(jnp.where(g > 0, g, 0.2 * g))
                    vale[i, sl] = expe
                    vales[i, sl] = expe * st
                    valew[i, sl] = w * st
                    valf[i, sl] = expf
                    valg[i, sl] = expg
                return c2
            lax.fori_loop(0, 16, row, 0)

            pltpu.sync_copy(valf, expfg_h.at[pl.ds(rb, 16), :])
            pltpu.sync_copy(valg, expfg_h.at[pl.ds(G + rb, 16), :])
            pend = []
            for i in range(16):
                pend.append([
                    pltpu.async_copy(vale.at[i], acc_e.at[dst2.at[i]], sem, add=True),
                    pltpu.async_copy(vales.at[i], acc_es.at[dst2.at[i]], sem, add=True),
                    pltpu.async_copy(valew.at[i], acc_ew.at[dst2.at[i]], sem, add=True),
                    pltpu.async_copy(valf.at[i], acc_f.at[src2.at[i]], sem, add=True),
                    pltpu.async_copy(valg.at[i], acc_g.at[dst2.at[i]], sem, add=True),
                ])
                if i >= 2:
                    for dd in pend[i - 2]:
                        dd.wait()
            for row_d in pend[14:]:
                for dd in row_d:
                    dd.wait()
            return carry
        lax.fori_loop(0, CHA, chunk, 0)
        plsc.subcore_barrier()
        for ai in range(5):
            @pl.when(s == ai)
            def _(ai=ai):
                pltpu.sync_copy(accs[ai], stage)
                pltpu.sync_copy(stage, sums_h.at[pl.ds((c * 5 + ai) * NPAD, NPAD)])

    return passA


def _make_passW(NPAD, G):
    RB = G // 16
    CHB = RB // 16
    mesh = plsc.VectorSubcoreMesh(core_axis_name="c", subcore_axis_name="s")

    @functools.partial(
        pl.kernel,
        out_type=jax.ShapeDtypeStruct((2 * G * 128,), f32),
        mesh=mesh,
        compiler_params=pltpu.CompilerParams(needs_layout_passes=False),
        scratch_types=[
            pltpu.VMEM((NPAD,), f32),
            pltpu.VMEM((32, 64), i32),
            pltpu.VMEM((16, 128), f32),
            pltpu.VMEM((16, 128), f32),
            pltpu.VMEM((2048,), f32),
            pltpu.VMEM((16,), f32),
            pltpu.VMEM((16,), f32),
        ],
    )
    def passW(sidx_h, ew_h, exp_h, sinv_h, wa_h, wb_h, alph_h,
              sinv_v, sidx2, ew2, exp2, alpha_f, wa_v, wb_v):
        c = lax.axis_index("c")
        s = lax.axis_index("s")
        pltpu.sync_copy(wa_h.at[pl.ds(c * 16, 16)], wa_v)
        pltpu.sync_copy(wb_h.at[pl.ds(c * 16, 16)], wb_v)
        pltpu.sync_copy(sinv_h.at[pl.ds(c * NPAD, NPAD)], sinv_v)
        wa = wa_v[...]
        wb = wb_v[...]
        base = s * RB
        cbase = c * G + base

        def chunk0(ch, carry):
            pltpu.sync_copy(ew_h.at[pl.ds(base + ch * 16, 16), :], ew2)
            pltpu.sync_copy(exp_h.at[pl.ds(cbase + ch * 16, 16), :], exp2)
            pltpu.sync_copy(sidx_h.at[pl.ds((cbase + ch * 16) * 2, 32), :],
                            sidx2)

            def arow(i, c2):
                for k in range(8):
                    sl = pl.ds(k * 16, 16)
                    sg = plsc.load_gather(
                        sinv_v, [sidx2[2 * i + k // 4, pl.ds(16 * (k % 4), 16)]])
                    alpha_f[pl.ds(i * 128 + k * 16, 16)] = (
                        exp2[i, sl] * sg * wa + ew2[i, sl] * wb)
                return c2
            lax.fori_loop(0, 16, arow, 0)
            pltpu.sync_copy(alpha_f,
                            alph_h.at[pl.ds((cbase + ch * 16) * 128, 2048)])
            return carry
        lax.fori_loop(0, CHB, chunk0, 0)

    return passW

def _make_passB(NPAD, G):
    RB = G // 16
    CHB = RB // 16
    ZR = NPAD // 16
    ZFULL, ZREM = ZR // 128, ZR % 128
    mesh = plsc.VectorSubcoreMesh(core_axis_name="c", subcore_axis_name="s")

    @functools.partial(
        pl.kernel,
        out_type=jax.ShapeDtypeStruct((2, NPAD, H), f32),
        mesh=mesh,
        compiler_params=pltpu.CompilerParams(needs_layout_passes=False),
        scratch_types=[
            pltpu.VMEM((32, 64), i32),
            pltpu.VMEM((32, 64), i32),
            pltpu.VMEM((2048,), f32),
            pltpu.VMEM((64, H), f32),
            pltpu.VMEM((64, H), f32),
            pltpu.VMEM((64, H), f32),
            pltpu.VMEM((64, H), f32),
            pltpu.VMEM_SHARED((NPAD, H), f32),
            pltpu.SemaphoreType.DMA,
            pltpu.SemaphoreType.DMA,
            pltpu.SemaphoreType.DMA,
            pltpu.SemaphoreType.DMA,
            pltpu.SemaphoreType.DMA,
            pltpu.SemaphoreType.DMA,
            pltpu.SemaphoreType.DMA,
            pltpu.SemaphoreType.DMA,
        ],
    )
    def passB(gidx_h, sidx_h, alph_h, gate_h, out_h,
              gidx2, sidx2, alpha_f, g0, g1, g2, g3, acc,
              sg0, sg1, sg2, sg3, ss0, ss1, ss2, ss3):
        c = lax.axis_index("c")
        s = lax.axis_index("s")
        gbuf = [g0, g1, g2, g3]
        bbuf = gbuf
        sem_g = [sg0, sg1, sg2, sg3]
        sem_s = [ss0, ss1, ss2, ss3]
        ZR = NPAD // 16
        Z64 = ZR // 64

        def zr(k, carry):
            for v in range(8):
                g0[k, pl.ds(v * 16, 16)] = jnp.zeros((16,), f32)
            return carry
        lax.fori_loop(0, 64, zr, 0)
        zbase = s * ZR
        for kk in range(Z64):
            pltpu.sync_copy(g0, acc.at[pl.ds(zbase + kk * 64, 64), :])
        plsc.subcore_barrier()

        base = s * RB
        cbase = c * G + base

        bidx = [jnp.full((16,), q, i32) for q in range(16)]

        def scale(gb, bb, t):
            def grp(g, carry):
                v = alpha_f[pl.ds(t * 64 + g * 16, 16)]
                for q in range(16):
                    av = v.at[bidx[q]].get(mode="promise_in_bounds")
                    r = g * 16 + q
                    for vv in range(8):
                        slv = pl.ds(vv * 16, 16)
                        gb[r, slv] = gb[r, slv] * av
                return carry
            lax.fori_loop(0, 4, grp, 0)

        def gfire(t, b):
            pltpu.async_copy(gate_h.at[gidx2.at[t]], gbuf[b], sem_g[b])

        def gwait(t, b):
            pltpu.make_async_copy(
                gate_h.at[gidx2.at[t]], gbuf[b], sem_g[b]).wait()

        def sfire(t, b):
            pltpu.async_copy(bbuf[b], acc.at[sidx2.at[t]], sem_s[b],
                             add=True)

        def swait(b):
            pltpu.make_async_copy(
                bbuf[b], acc.at[sidx2.at[0]], sem_s[b]).wait()

        def chunk1(ch, carry):
            hb2 = (cbase + ch * 16) * 2
            pltpu.sync_copy(gidx_h.at[pl.ds(hb2, 32), :], gidx2)
            pltpu.sync_copy(sidx_h.at[pl.ds(hb2, 32), :], sidx2)
            pltpu.sync_copy(alph_h.at[pl.ds((cbase + ch * 16) * 128, 2048)],
                            alpha_f)
            gfire(0, 0)
            gfire(1, 1)

            def quad(tq, c2):
                t0 = tq * 4
                for b in range(4):
                    t = t0 + b

                    @pl.when((t >= 2) | (ch > 0))
                    def _():
                        swait((b + 2) % 4)

                    @pl.when(t + 2 < 32)
                    def _():
                        gfire(t + 2, (b + 2) % 4)
                    gwait(t, b)
                    scale(gbuf[b], bbuf[b], t)
                    sfire(t, b)
                return c2
            lax.fori_loop(0, 8, quad, 0)
            return carry
        lax.fori_loop(0, CHB, chunk1, 0)
        swait(2)
        swait(3)

        plsc.subcore_barrier()
        for kk in range(Z64):
            pltpu.sync_copy(acc.at[pl.ds(zbase + kk * 64, 64), :], g0)
            pltpu.sync_copy(g0, out_h.at[c, pl.ds(zbase + kk * 64, 64), :])

    return passB


def kernel(x, edge_index, edge_weight, trans_W, eta, sws, swn, swa, swe, beta, sgW1, sgb1, sgW2, sgb2, s_ws, s_wn, s_wst, s_wa, s_we, tau, tgW1, tgb1, tgW2, tgb2, t_ws, t_wn, t_wst, t_wa, t_we, theta1, theta2, theta3, theta4):
    n = x.shape[0]
    e = edge_index.shape[1]
    NPAD = ((n + 1 + 255) // 256) * 256
    EP = ((e + 65535) // 65536) * 65536
    G = EP // 128
    NB = 10
    BR = n // NB

    src = edge_index[0]
    dst = edge_index[1]
    seed = x[:, -1]
    source = x[:, :H]
    target = x[:, H:2 * H]

    pad_i = jnp.full((EP - e,), n, i32)
    srcR = jnp.concatenate([src, pad_i]).reshape(G, 128)
    dstR = jnp.concatenate([dst, pad_i]).reshape(G, 128)
    ewR = jnp.concatenate([edge_weight, jnp.zeros((EP - e,), f32)]).reshape(G, 128)

    passA = _make_passA(NPAD, G)
    passW = _make_passW(NPAD, G)
    passB = _make_passB(NPAD, G)

    k1 = pl.pallas_call(
        _k1_body,
        out_shape=(jax.ShapeDtypeStruct((n, H), f32),) * 4
        + (jax.ShapeDtypeStruct((n, 8), f32),),
        grid=(NB,),
        in_specs=[
            pl.BlockSpec((BR, H), lambda i: (i, 0)),
            pl.BlockSpec((BR, H), lambda i: (i, 0)),
            pl.BlockSpec((BR, 1), lambda i: (i, 0)),
        ] + [pl.BlockSpec((H, H), lambda i: (0, 0)),
             pl.BlockSpec((H, 8), lambda i: (0, 0)),
             pl.BlockSpec((H, 8), lambda i: (0, 0)),
             pl.BlockSpec((H, H), lambda i: (0, 0)),
             pl.BlockSpec((1, H), lambda i: (0, 0)),
             pl.BlockSpec((H, H), lambda i: (0, 0)),
             pl.BlockSpec((1, H), lambda i: (0, 0)),
             pl.BlockSpec((H, H), lambda i: (0, 0)),
             pl.BlockSpec((1, H), lambda i: (0, 0)),
             pl.BlockSpec((H, H), lambda i: (0, 0)),
             pl.BlockSpec((1, H), lambda i: (0, 0))],
        out_specs=[pl.BlockSpec((BR, H), lambda i: (i, 0))] * 4
        + [pl.BlockSpec((BR, 8), lambda i: (i, 0))],
    )

    k2 = pl.pallas_call(
        _k2_body,
        out_shape=(jax.ShapeDtypeStruct((1, NPAD), f32),
                   jax.ShapeDtypeStruct((2, NPAD), f32)),
    )

    _pre_specs = [pl.BlockSpec((BR, H), lambda i: (i, 0))] * 4 + [
        pl.BlockSpec((BR, 1), lambda i: (i, 0)),
        pl.BlockSpec((1, 8), lambda i: (0, 0))]

    k1f = pl.pallas_call(
        _k1f_body,
        out_shape=(jax.ShapeDtypeStruct((n, H), f32),) * 4
        + (jax.ShapeDtypeStruct((n, 8), f32),),
        grid=(NB,),
        in_specs=_pre_specs
        + [pl.BlockSpec((H, H), lambda i: (0, 0)),
           pl.BlockSpec((H, 8), lambda i: (0, 0)),
           pl.BlockSpec((H, 8), lambda i: (0, 0)),
           pl.BlockSpec((H, H), lambda i: (0, 0)),
           pl.BlockSpec((1, H), lambda i: (0, 0)),
           pl.BlockSpec((H, H), lambda i: (0, 0)),
           pl.BlockSpec((1, H), lambda i: (0, 0)),
           pl.BlockSpec((H, H), lambda i: (0, 0)),
           pl.BlockSpec((1, H), lambda i: (0, 0)),
           pl.BlockSpec((H, H), lambda i: (0, 0)),
           pl.BlockSpec((1, H), lambda i: (0, 0))],
        out_specs=[pl.BlockSpec((BR, H), lambda i: (i, 0))] * 4
        + [pl.BlockSpec((BR, 8), lambda i: (i, 0))],
    )

    kff = pl.pallas_call(
        _kff_body,
        out_shape=jax.ShapeDtypeStruct((n, 1), f32),
        grid=(NB,),
        in_specs=_pre_specs
        + [pl.BlockSpec((H, H), lambda i: (0, 0)),
           pl.BlockSpec((H, H), lambda i: (0, 0)),
           pl.BlockSpec((H, H), lambda i: (0, 0)),
           pl.BlockSpec((3 * H, 1), lambda i: (0, 0))],
        out_specs=pl.BlockSpec((BR, 1), lambda i: (i, 0)),
    )

    zrow = jnp.zeros((NPAD - n, H), f32)
    seed_p = jnp.concatenate([seed, jnp.zeros((NPAD - n,), f32)]).reshape(1, NPAD)
    gidxM = jnp.concatenate([dstR, srcR + NPAD], axis=0).reshape(-1, 64)
    sidxM = jnp.concatenate([srcR, dstR], axis=0).reshape(-1, 64)

    def edge_phase(i, proj, sgate, tgate, state_p):
        ns = jnp.concatenate([proj, jnp.zeros((NPAD - n, 8), f32)], axis=0).reshape(NPAD * 8)
        sums, expfgR = passA(srcR, dstR, ewR, ns)
        par2 = jnp.stack([sws[i], swn[i], swa[i], swe[i],
                          jnp.zeros((), f32), jnp.zeros((), f32),
                          jnp.zeros((), f32), jnp.zeros((), f32)])[None, :]
        nst, sinv = k2(sums.reshape(2, 5, NPAD), state_p, seed_p, par2)
        wa2 = jnp.concatenate([jnp.full((16,), s_wa[i], f32),
                               jnp.full((16,), t_wa[i], f32)])
        wb2 = jnp.concatenate([jnp.full((16,), s_we[i], f32),
                               jnp.full((16,), t_we[i], f32)])
        gate32 = jnp.concatenate([sgate, zrow, tgate, zrow], axis=0)
        alph = passW(sidxM, ewR, expfgR, sinv.reshape(2 * NPAD), wa2, wb2)
        bc = passB(gidxM, sidxM, alph, gate32)
        par3 = jnp.stack([s_ws[i], s_wn[i], s_wst[i],
                          t_ws[i], t_wn[i], t_wst[i],
                          jnp.zeros((), f32), jnp.zeros((), f32)])[None, :]
        return nst, bc, par3

    def weights(i):
        a8 = jnp.concatenate(
            [eta[i][:H], beta[i][:H], tau[i][:H], jnp.zeros((H, 5), f32)], axis=1)
        b8 = jnp.concatenate(
            [jnp.zeros((H, 3), f32), eta[i][H:], beta[i][H:], tau[i][H:],
             jnp.zeros((H, 2), f32)], axis=1)
        return [trans_W[i], a8, b8,
                sgW1[i].T, sgb1[i][None, :], sgW2[i].T, sgb2[i][None, :],
                tgW1[i].T, tgb1[i][None, :], tgW2[i].T, tgb2[i][None, :]]

    tsi, tti, sgate, tgate, proj = k1(source, target, seed[:, None], *weights(0))
    nst, bc, par3 = edge_phase(0, proj, sgate, tgate, seed_p)

    tsi, tti, sgate, tgate, proj = k1f(
        tsi, tti, bc[0, :n], bc[1, :n], nst[0, :n, None], par3, *weights(1))
    nst, bc, par3 = edge_phase(1, proj, sgate, tgate, nst[:, :NPAD])

    q = kff(tsi, tti, bc[0, :n], bc[1, :n], nst[0, :n, None], par3,
            theta2, theta3, theta4, theta1)
    return q[:, 0]
